# Initial kernel scaffold; baseline (speedup 1.0000x reference)
import jax
import jax.numpy as jnp
from jax.experimental import pallas as pl


def kernel(positions, species, senders, receivers, species_embed, W_rad1, W_rad2, W_upd, W_read):
    raise NotImplementedError("write your pallas kernel here")


# speedup vs baseline: 2.3327x; 2.3327x over previous
import functools

import numpy as np
import jax
import jax.numpy as jnp
from jax import lax
from jax.experimental import pallas as pl
from jax.experimental.pallas import tpu as pltpu
from jax.experimental.pallas import tpu_sc as plsc

_N = 10000
_E = 320000
_C = 32
_NBAS = 8
_RMAX = 5.0
_LMAX = 3
_T = 2
_OUT = 128
_HID = 64
_AVG = 32.0
_NLM = 16

_L_OF = np.array([0, 1, 1, 1, 2, 2, 2, 2, 2, 3, 3, 3, 3, 3, 3, 3], np.int32)

_EB = 512
_NB = 1000


def _silu(x):
    return x / (1.0 + jnp.exp(-x))


def _geom_body(ps_ref, pr_ref, rb_ref, sh_ref):
    ps = ps_ref[...]
    pr = pr_ref[...]
    rel = pr - ps
    d2 = jnp.sum(rel * rel, axis=1, keepdims=True)
    dist = jnp.sqrt(d2)
    inv = 1.0 / jnp.maximum(dist, 1e-6)
    x = rel[:, 0:1] * inv
    y = rel[:, 1:2] * inv
    z = rel[:, 2:3] * inv

    ks = (jnp.arange(_NBAS, dtype=jnp.int32).astype(jnp.float32) + 1.0)[None, :]
    d_safe = jnp.where(dist == 0.0, 1e-6, dist)
    rb = jnp.sqrt(2.0 / _RMAX) * jnp.sin(ks * (jnp.pi / _RMAX) * d_safe) / d_safe
    u = dist * (1.0 / _RMAX)
    u2 = u * u
    env = jnp.exp(-u2 / jnp.clip(1.0 - u2, 1e-6, None))
    env = jnp.where(u < 1.0, env, 0.0)
    rb_ref[...] = rb * env

    one = jnp.ones_like(x)
    x2 = x * x
    y2 = y * y
    z2 = z * z
    cols = [
        0.28209479 * one,
        0.48860251 * y,
        0.48860251 * z,
        0.48860251 * x,
        1.09254843 * x * y,
        1.09254843 * y * z,
        0.31539157 * (3.0 * z2 - 1.0),
        1.09254843 * x * z,
        0.54627422 * (x2 - y2),
        0.59004359 * y * (3.0 * x2 - y2),
        2.89061144 * x * y * z,
        0.45704580 * y * (5.0 * z2 - 1.0),
        0.37317633 * z * (5.0 * z2 - 3.0),
        0.45704580 * x * (5.0 * z2 - 1.0),
        1.44530572 * z * (x2 - y2),
        0.59004359 * x * (x2 - 3.0 * y2),
    ]
    sh_ref[...] = jnp.concatenate(cols, axis=1)


def _geom(ps, pr):
    grid = _E // _EB
    return pl.pallas_call(
        _geom_body,
        grid=(grid,),
        in_specs=[
            pl.BlockSpec((_EB, 8), lambda i: (i, 0)),
            pl.BlockSpec((_EB, 8), lambda i: (i, 0)),
        ],
        out_specs=[
            pl.BlockSpec((_EB, _NBAS), lambda i: (i, 0)),
            pl.BlockSpec((_EB, _NLM), lambda i: (i, 0)),
        ],
        out_shape=[
            jax.ShapeDtypeStruct((_E, _NBAS), jnp.float32),
            jax.ShapeDtypeStruct((_E, _NLM), jnp.float32),
        ],
    )(ps, pr)


def _edge_body(rb_ref, sh_ref, hs_ref, w1_ref, w2_ref, out_ref):
    rb = rb_ref[...]
    sh = sh_ref[...]
    hs = hs_ref[...]
    hid = _silu(jnp.dot(rb, w1_ref[...], preferred_element_type=jnp.float32))
    rad = jnp.dot(hid, w2_ref[...], preferred_element_type=jnp.float32)
    rad3 = rad.reshape(_EB, _C, _LMAX + 1)
    radlm = jnp.concatenate(
        [
            jnp.broadcast_to(rad3[:, :, l : l + 1], (_EB, _C, 2 * l + 1))
            for l in range(_LMAX + 1)
        ],
        axis=2,
    )
    msg = radlm * hs[:, :, None] * sh[:, None, :]
    for k in range(4):
        out_ref[k] = msg[:, 8 * k : 8 * (k + 1), :].reshape(_EB, 128)


def _edge(rb, sh, hs, w1, w2):
    grid = _E // _EB
    return pl.pallas_call(
        _edge_body,
        grid=(grid,),
        in_specs=[
            pl.BlockSpec((_EB, _NBAS), lambda i: (i, 0)),
            pl.BlockSpec((_EB, _NLM), lambda i: (i, 0)),
            pl.BlockSpec((_EB, _C), lambda i: (i, 0)),
            pl.BlockSpec((_NBAS, _HID), lambda i: (0, 0)),
            pl.BlockSpec((_HID, _C * (_LMAX + 1)), lambda i: (0, 0)),
        ],
        out_specs=pl.BlockSpec((4, _EB, 128), lambda i: (0, i, 0)),
        out_shape=jax.ShapeDtypeStruct((4, _E, 128), jnp.float32),
    )(rb, sh, hs, w1, w2)


def _node_body(agg_ref, mw_ref, wr_ref, h_ref, rd_ref):
    h_pre = jnp.zeros((_NB, _C), jnp.float32)
    for k in range(4):
        a = agg_ref[k] * (1.0 / _AVG)
        sq = a * a
        h_pre = h_pre + jnp.dot(sq, mw_ref[k], preferred_element_type=jnp.float32)
    h_new = _silu(h_pre)
    h_ref[...] = h_new
    rd_ref[...] = jnp.dot(h_new, wr_ref[...], preferred_element_type=jnp.float32)


def _node(agg4, mw, w_read):
    grid = _N // _NB
    return pl.pallas_call(
        _node_body,
        grid=(grid,),
        in_specs=[
            pl.BlockSpec((4, _NB, 128), lambda i: (0, i, 0)),
            pl.BlockSpec((4, 128, _C), lambda i: (0, 0, 0)),
            pl.BlockSpec((_C, _OUT), lambda i: (0, 0)),
        ],
        out_specs=[
            pl.BlockSpec((_NB, _C), lambda i: (i, 0)),
            pl.BlockSpec((_NB, _OUT), lambda i: (i, 0)),
        ],
        out_shape=[
            jax.ShapeDtypeStruct((_N, _C), jnp.float32),
            jax.ShapeDtypeStruct((_N, _OUT), jnp.float32),
        ],
    )(agg4, mw, w_read)


def _make_mw(w_upd_t):
    rows = np.zeros((4, 128), np.int32)
    for k in range(4):
        for c8 in range(8):
            for lm in range(_NLM):
                rows[k, c8 * 16 + lm] = _L_OF[lm] * _C + 8 * k + c8
    return w_upd_t[rows.reshape(-1)].reshape(4, 128, _C)


def kernel(positions, species, senders, receivers, species_embed, W_rad1, W_rad2, W_upd, W_read):
    pos_pad = jnp.pad(positions, ((0, 0), (0, 5)))
    ps = pos_pad[senders]
    pr = pos_pad[receivers]
    rb, sh = _geom(ps, pr)

    h = species_embed[species]
    reads = []
    for t in range(_T):
        hs = h[senders]
        msg4 = _edge(rb, sh, hs, W_rad1[t], W_rad2[t])
        agg4 = jax.vmap(
            lambda m: jax.ops.segment_sum(m, receivers, num_segments=_N)
        )(msg4)
        h, rd = _node(agg4, _make_mw(W_upd[t]), W_read[t])
        reads.append(rd)
    return jnp.concatenate(reads, axis=1)


# speedup vs baseline: 9.4765x; 4.0624x over previous
import functools

import numpy as np
import jax
import jax.numpy as jnp
from jax import lax
from jax.experimental import pallas as pl
from jax.experimental.pallas import tpu as pltpu
from jax.experimental.pallas import tpu_sc as plsc

_N = 10000
_E = 320000
_C = 32
_NBAS = 8
_RMAX = 5.0
_LMAX = 3
_T = 2
_OUT = 128
_HID = 64
_AVG = 32.0
_NLM = 16

_L_OF = np.array([0, 1, 1, 1, 2, 2, 2, 2, 2, 3, 3, 3, 3, 3, 3, 3], np.int32)

_EB = 512
_NB = 1000


def _silu(x):
    return x / (1.0 + jnp.exp(-x))


def _geom_body(ps_ref, pr_ref, rb_ref, sh_ref):
    ps = ps_ref[...]
    pr = pr_ref[...]
    rel = pr - ps
    d2 = jnp.sum(rel * rel, axis=1, keepdims=True)
    dist = jnp.sqrt(d2)
    inv = 1.0 / jnp.maximum(dist, 1e-6)
    x = rel[:, 0:1] * inv
    y = rel[:, 1:2] * inv
    z = rel[:, 2:3] * inv

    ks = (jnp.arange(_NBAS, dtype=jnp.int32).astype(jnp.float32) + 1.0)[None, :]
    d_safe = jnp.where(dist == 0.0, 1e-6, dist)
    rb = jnp.sqrt(2.0 / _RMAX) * jnp.sin(ks * (jnp.pi / _RMAX) * d_safe) / d_safe
    u = dist * (1.0 / _RMAX)
    u2 = u * u
    env = jnp.exp(-u2 / jnp.clip(1.0 - u2, 1e-6, None))
    env = jnp.where(u < 1.0, env, 0.0)
    rb_ref[...] = rb * env

    one = jnp.ones_like(x)
    x2 = x * x
    y2 = y * y
    z2 = z * z
    cols = [
        0.28209479 * one,
        0.48860251 * y,
        0.48860251 * z,
        0.48860251 * x,
        1.09254843 * x * y,
        1.09254843 * y * z,
        0.31539157 * (3.0 * z2 - 1.0),
        1.09254843 * x * z,
        0.54627422 * (x2 - y2),
        0.59004359 * y * (3.0 * x2 - y2),
        2.89061144 * x * y * z,
        0.45704580 * y * (5.0 * z2 - 1.0),
        0.37317633 * z * (5.0 * z2 - 3.0),
        0.45704580 * x * (5.0 * z2 - 1.0),
        1.44530572 * z * (x2 - y2),
        0.59004359 * x * (x2 - 3.0 * y2),
    ]
    sh_ref[...] = jnp.concatenate(cols, axis=1)


def _geom(ps, pr):
    grid = _E // _EB
    return pl.pallas_call(
        _geom_body,
        grid=(grid,),
        in_specs=[
            pl.BlockSpec((_EB, 16), lambda i: (i, 0)),
            pl.BlockSpec((_EB, 16), lambda i: (i, 0)),
        ],
        out_specs=[
            pl.BlockSpec((_EB, _NBAS), lambda i: (i, 0)),
            pl.BlockSpec((_EB, _NLM), lambda i: (i, 0)),
        ],
        out_shape=[
            jax.ShapeDtypeStruct((_E, _NBAS), jnp.float32),
            jax.ShapeDtypeStruct((_E, _NLM), jnp.float32),
        ],
    )(ps, pr)


def _edge_body(rb_ref, sh_ref, hs_ref, w1_ref, w2_ref, out_ref):
    rb = rb_ref[...]
    sh = sh_ref[...]
    hs = hs_ref[...]
    hid = _silu(jnp.dot(rb, w1_ref[...], preferred_element_type=jnp.float32))
    rad = jnp.dot(hid, w2_ref[...], preferred_element_type=jnp.float32)
    rad3 = rad.reshape(_EB, _C, _LMAX + 1)
    radlm = jnp.concatenate(
        [
            jnp.broadcast_to(rad3[:, :, l : l + 1], (_EB, _C, 2 * l + 1))
            for l in range(_LMAX + 1)
        ],
        axis=2,
    )
    msg = radlm * hs[:, :, None] * sh[:, None, :]
    for k in range(4):
        out_ref[k] = msg[:, 8 * k : 8 * (k + 1), :].reshape(_EB, 128)


def _edge(rb, sh, hs, w1, w2):
    grid = _E // _EB
    return pl.pallas_call(
        _edge_body,
        grid=(grid,),
        in_specs=[
            pl.BlockSpec((_EB, _NBAS), lambda i: (i, 0)),
            pl.BlockSpec((_EB, _NLM), lambda i: (i, 0)),
            pl.BlockSpec((_EB, _C), lambda i: (i, 0)),
            pl.BlockSpec((_NBAS, _HID), lambda i: (0, 0)),
            pl.BlockSpec((_HID, _C * (_LMAX + 1)), lambda i: (0, 0)),
        ],
        out_specs=pl.BlockSpec((4, _EB, 128), lambda i: (0, i, 0)),
        out_shape=jax.ShapeDtypeStruct((4, _E, 128), jnp.float32),
    )(rb, sh, hs, w1, w2)


def _node_body(agg_ref, mw_ref, wr_ref, h_ref, rd_ref):
    h_pre = jnp.zeros((_NB, _C), jnp.float32)
    for k in range(4):
        a = agg_ref[k] * (1.0 / _AVG)
        sq = a * a
        h_pre = h_pre + jnp.dot(sq, mw_ref[k], preferred_element_type=jnp.float32)
    h_new = _silu(h_pre)
    h_ref[...] = h_new
    rd_ref[...] = jnp.dot(h_new, wr_ref[...], preferred_element_type=jnp.float32)


def _node(agg4, mw, w_read):
    grid = _N // _NB
    return pl.pallas_call(
        _node_body,
        grid=(grid,),
        in_specs=[
            pl.BlockSpec((4, _NB, 128), lambda i: (0, i, 0)),
            pl.BlockSpec((4, 128, _C), lambda i: (0, 0, 0)),
            pl.BlockSpec((_C, _OUT), lambda i: (0, 0)),
        ],
        out_specs=[
            pl.BlockSpec((_NB, _C), lambda i: (i, 0)),
            pl.BlockSpec((_NB, _OUT), lambda i: (i, 0)),
        ],
        out_shape=[
            jax.ShapeDtypeStruct((_N, _C), jnp.float32),
            jax.ShapeDtypeStruct((_N, _OUT), jnp.float32),
        ],
    )(agg4, mw, w_read)


_GB = 80
_NW = 32


def _sc_mesh():
    return plsc.VectorSubcoreMesh(
        core_axis_name="c", subcore_axis_name="s", num_cores=2, num_subcores=16
    )


def _sc_gather(table, idx2d, rows, width):
    per_w = rows // _NW
    nb = per_w // _GB

    @functools.partial(
        pl.kernel,
        mesh=_sc_mesh(),
        out_type=jax.ShapeDtypeStruct((rows, width), jnp.float32),
        scratch_types=[
            pltpu.VMEM((nb, _GB), jnp.int32),
            pltpu.VMEM((_GB, width), jnp.float32),
            pltpu.VMEM((_GB, width), jnp.float32),
            pltpu.SemaphoreType.DMA,
            pltpu.SemaphoreType.DMA,
        ],
        compiler_params=pltpu.CompilerParams(use_tc_tiling_on_sc=False),
    )
    def k(table_hbm, idx_hbm, out_hbm, idx_v, buf0, buf1, sem0, sem1):
        cid = lax.axis_index("c")
        sid = lax.axis_index("s")
        wid = sid * 2 + cid
        base = wid * per_w
        pltpu.sync_copy(idx_hbm.at[pl.ds(wid * nb, nb)], idx_v)

        def body(j, _):
            pltpu.async_copy(table_hbm.at[idx_v.at[j]], buf0, sem0).wait()
            pltpu.sync_copy(buf0, out_hbm.at[pl.ds(base + j * _GB, _GB)])
            return 0

        lax.fori_loop(0, nb, body, 0)

    return k(table, idx2d)


def _sc_scatter_add(msg_flat, recv2d):
    per_tile = _E // 16
    nb = per_tile // _GB
    nrow_t = _N // 16

    def _zero_acc(zsrc, acc, nbase):
        for q in range(7):
            pltpu.sync_copy(zsrc, acc.at[pl.ds(nbase + q * _GB, _GB)])
        pltpu.sync_copy(zsrc.at[pl.ds(0, 65)], acc.at[pl.ds(nbase + 7 * _GB, 65)])

    @functools.partial(
        pl.kernel,
        mesh=_sc_mesh(),
        out_type=jax.ShapeDtypeStruct((4 * _N, 128), jnp.float32),
        scratch_types=[
            pltpu.VMEM((nb, _GB), jnp.int32),
            pltpu.VMEM((_GB, 128), jnp.float32),
            pltpu.VMEM((_GB, 128), jnp.float32),
            pltpu.VMEM_SHARED((_N, 128), jnp.float32),
            pltpu.SemaphoreType.DMA,
            pltpu.SemaphoreType.DMA,
        ],
        compiler_params=pltpu.CompilerParams(use_tc_tiling_on_sc=False),
    )
    def k(msg_hbm, idx_hbm, zer_hbm, out_hbm, idx_v, buf0, buf1, acc, sem0, sem1):
        cid = lax.axis_index("c")
        sid = lax.axis_index("s")
        ebase = sid * per_tile
        nbase = sid * nrow_t
        pltpu.sync_copy(idx_hbm.at[pl.ds(sid * nb, nb)], idx_v)
        pltpu.sync_copy(zer_hbm, buf0)
        _zero_acc(buf0, acc, nbase)
        plsc.subcore_barrier()
        for chunk in range(2):
            cglob = cid * 2 + chunk
            row0 = cglob * _E + ebase

            def body(j, _):
                pltpu.async_copy(
                    msg_hbm.at[pl.ds(row0 + j * _GB, _GB)], buf1, sem1
                ).wait()
                pltpu.sync_copy(buf1, acc.at[idx_v.at[j]], add=True)
                return 0

            lax.fori_loop(0, nb, body, 0)
            plsc.subcore_barrier()
            pltpu.sync_copy(
                acc.at[pl.ds(nbase, nrow_t)],
                out_hbm.at[pl.ds(cglob * _N + nbase, nrow_t)],
            )
            if chunk == 0:
                pltpu.sync_copy(zer_hbm, buf0)
                _zero_acc(buf0, acc, nbase)
                plsc.subcore_barrier()

    zer = jnp.zeros((_GB, 128), jnp.float32)
    return k(msg_flat, recv2d, zer)


def _make_mw(w_upd_t):
    rows = np.zeros((4, 128), np.int32)
    for k in range(4):
        for c8 in range(8):
            for lm in range(_NLM):
                rows[k, c8 * 16 + lm] = _L_OF[lm] * _C + 8 * k + c8
    return w_upd_t[rows.reshape(-1)].reshape(4, 128, _C)


def kernel(positions, species, senders, receivers, species_embed, W_rad1, W_rad2, W_upd, W_read):
    pos_pad = jnp.pad(positions, ((0, 0), (0, 13)))
    send2d = senders.astype(jnp.int32).reshape(_E // _GB, _GB)
    recv2d = receivers.astype(jnp.int32).reshape(_E // _GB, _GB)

    ps = _sc_gather(pos_pad, send2d, _E, 16)
    pr = _sc_gather(pos_pad, recv2d, _E, 16)
    rb, sh = _geom(ps, pr)

    npad = 10240
    spec2d = jnp.pad(species.astype(jnp.int32), (0, npad - _N)).reshape(npad // _GB, _GB)
    h = _sc_gather(species_embed, spec2d, npad, _C)[:_N]

    reads = []
    for t in range(_T):
        hs = _sc_gather(h, send2d, _E, _C)
        msg4 = _edge(rb, sh, hs, W_rad1[t], W_rad2[t])
        agg4 = _sc_scatter_add(msg4.reshape(4 * _E, 128), recv2d).reshape(4, _N, 128)
        h, rd = _node(agg4, _make_mw(W_upd[t]), W_read[t])
        reads.append(rd)
    return jnp.concatenate(reads, axis=1)


# speedup vs baseline: 43.1725x; 4.5557x over previous
import functools

import numpy as np
import jax
import jax.numpy as jnp
from jax import lax
from jax.experimental import pallas as pl
from jax.experimental.pallas import tpu as pltpu
from jax.experimental.pallas import tpu_sc as plsc

_N = 10000
_E = 320000
_C = 32
_NBAS = 8
_RMAX = 5.0
_LMAX = 3
_T = 2
_OUT = 128
_HID = 64
_AVG = 32.0
_NLM = 16

_L_OF = np.array([0, 1, 1, 1, 2, 2, 2, 2, 2, 3, 3, 3, 3, 3, 3, 3], np.int32)

_EB = 512
_NB = 1000


def _silu(x):
    return x / (1.0 + jnp.exp(-x))


def _geom_body(ps_ref, pr_ref, rb_ref, sh_ref):
    ps = ps_ref[...]
    pr = pr_ref[...]
    rel = pr - ps
    d2 = jnp.sum(rel * rel, axis=1, keepdims=True)
    dist = jnp.sqrt(d2)
    inv = 1.0 / jnp.maximum(dist, 1e-6)
    x = rel[:, 0:1] * inv
    y = rel[:, 1:2] * inv
    z = rel[:, 2:3] * inv

    ks = (jnp.arange(_NBAS, dtype=jnp.int32).astype(jnp.float32) + 1.0)[None, :]
    d_safe = jnp.where(dist == 0.0, 1e-6, dist)
    rb = jnp.sqrt(2.0 / _RMAX) * jnp.sin(ks * (jnp.pi / _RMAX) * d_safe) / d_safe
    u = dist * (1.0 / _RMAX)
    u2 = u * u
    env = jnp.exp(-u2 / jnp.clip(1.0 - u2, 1e-6, None))
    env = jnp.where(u < 1.0, env, 0.0)
    rb_ref[...] = rb * env

    one = jnp.ones_like(x)
    x2 = x * x
    y2 = y * y
    z2 = z * z
    cols = [
        0.28209479 * one,
        0.48860251 * y,
        0.48860251 * z,
        0.48860251 * x,
        1.09254843 * x * y,
        1.09254843 * y * z,
        0.31539157 * (3.0 * z2 - 1.0),
        1.09254843 * x * z,
        0.54627422 * (x2 - y2),
        0.59004359 * y * (3.0 * x2 - y2),
        2.89061144 * x * y * z,
        0.45704580 * y * (5.0 * z2 - 1.0),
        0.37317633 * z * (5.0 * z2 - 3.0),
        0.45704580 * x * (5.0 * z2 - 1.0),
        1.44530572 * z * (x2 - y2),
        0.59004359 * x * (x2 - 3.0 * y2),
    ]
    sh_ref[...] = jnp.concatenate(cols, axis=1)


def _geom(ps, pr):
    grid = _E // _EB
    return pl.pallas_call(
        _geom_body,
        grid=(grid,),
        in_specs=[
            pl.BlockSpec((_EB, 16), lambda i: (i, 0)),
            pl.BlockSpec((_EB, 16), lambda i: (i, 0)),
        ],
        out_specs=[
            pl.BlockSpec((_EB, _NBAS), lambda i: (i, 0)),
            pl.BlockSpec((_EB, _NLM), lambda i: (i, 0)),
        ],
        out_shape=[
            jax.ShapeDtypeStruct((_E, _NBAS), jnp.float32),
            jax.ShapeDtypeStruct((_E, _NLM), jnp.float32),
        ],
    )(ps, pr)


_EBE = 1280


def _edge_body(rb_ref, sh_ref, hs_ref, w1_ref, w2p_ref, q_ref, out_ref):
    rb = rb_ref[...]
    sh = sh_ref[...]
    hs = hs_ref[...]
    hid = _silu(jnp.dot(rb, w1_ref[...], preferred_element_type=jnp.float32))
    radlm = jnp.dot(hid, w2p_ref[...], preferred_element_type=jnp.float32)
    hsx = jnp.dot(hs, q_ref[...], preferred_element_type=jnp.float32)
    shx = jnp.concatenate([sh] * 8, axis=1)
    msg = radlm * hsx
    for k in range(4):
        out_ref[k] = msg[:, 128 * k : 128 * (k + 1)] * shx


def _edge(rb, sh, hs, w1, w2p, q):
    grid = _E // _EBE
    return pl.pallas_call(
        _edge_body,
        grid=(grid,),
        in_specs=[
            pl.BlockSpec((_EBE, _NBAS), lambda i: (i, 0)),
            pl.BlockSpec((_EBE, _NLM), lambda i: (i, 0)),
            pl.BlockSpec((_EBE, _C), lambda i: (i, 0)),
            pl.BlockSpec((_NBAS, _HID), lambda i: (0, 0)),
            pl.BlockSpec((_HID, 512), lambda i: (0, 0)),
            pl.BlockSpec((_C, 512), lambda i: (0, 0)),
        ],
        out_specs=pl.BlockSpec((4, _EBE, 128), lambda i: (0, i, 0)),
        out_shape=jax.ShapeDtypeStruct((4, _E, 128), jnp.float32),
    )(rb, sh, hs, w1, w2p, q)


def _edge_weight_prep(w_rad2_t):
    colidx = np.zeros(512, np.int32)
    for k in range(4):
        for c8 in range(8):
            for lm in range(_NLM):
                colidx[128 * k + 16 * c8 + lm] = (8 * k + c8) * 4 + _L_OF[lm]
    return w_rad2_t[:, colidx]


def _make_q():
    q = np.zeros((_C, 512), np.float32)
    for k in range(4):
        for c8 in range(8):
            q[8 * k + c8, 128 * k + 16 * c8 : 128 * k + 16 * c8 + _NLM] = 1.0
    return jnp.asarray(q)


def _node_body(agg_ref, mw_ref, wr_ref, h_ref, rd_ref):
    h_pre = jnp.zeros((_NB, _C), jnp.float32)
    for k in range(4):
        a = agg_ref[k] * (1.0 / _AVG)
        sq = a * a
        h_pre = h_pre + jnp.dot(sq, mw_ref[k], preferred_element_type=jnp.float32)
    h_new = _silu(h_pre)
    h_ref[...] = h_new
    rd_ref[...] = jnp.dot(h_new, wr_ref[...], preferred_element_type=jnp.float32)


def _node(agg4, mw, w_read):
    grid = _N // _NB
    return pl.pallas_call(
        _node_body,
        grid=(grid,),
        in_specs=[
            pl.BlockSpec((4, _NB, 128), lambda i: (0, i, 0)),
            pl.BlockSpec((4, 128, _C), lambda i: (0, 0, 0)),
            pl.BlockSpec((_C, _OUT), lambda i: (0, 0)),
        ],
        out_specs=[
            pl.BlockSpec((_NB, _C), lambda i: (i, 0)),
            pl.BlockSpec((_NB, _OUT), lambda i: (i, 0)),
        ],
        out_shape=[
            jax.ShapeDtypeStruct((_N, _C), jnp.float32),
            jax.ShapeDtypeStruct((_N, _OUT), jnp.float32),
        ],
    )(agg4, mw, w_read)


_GB = 80
_NW = 32


def _sc_mesh():
    return plsc.VectorSubcoreMesh(
        core_axis_name="c", subcore_axis_name="s", num_cores=2, num_subcores=16
    )


def _sc_gather(table, idx2d, rows, width):
    per_w = rows // _NW
    nb = per_w // _GB

    @functools.partial(
        pl.kernel,
        mesh=_sc_mesh(),
        out_type=jax.ShapeDtypeStruct((rows, width), jnp.float32),
        scratch_types=[
            pltpu.VMEM((nb, _GB), jnp.int32),
            pltpu.VMEM((_GB, width), jnp.float32),
            pltpu.VMEM((_GB, width), jnp.float32),
            pltpu.SemaphoreType.DMA,
            pltpu.SemaphoreType.DMA,
        ],
        compiler_params=pltpu.CompilerParams(use_tc_tiling_on_sc=False),
    )
    def k(table_hbm, idx_hbm, out_hbm, idx_v, buf0, buf1, sem0, sem1):
        cid = lax.axis_index("c")
        sid = lax.axis_index("s")
        wid = sid * 2 + cid
        base = wid * per_w
        pltpu.sync_copy(idx_hbm.at[pl.ds(wid * nb, nb)], idx_v)

        def body(j, _):
            pltpu.async_copy(table_hbm.at[idx_v.at[j]], buf0, sem0).wait()
            pltpu.sync_copy(buf0, out_hbm.at[pl.ds(base + j * _GB, _GB)])
            return 0

        lax.fori_loop(0, nb, body, 0)

    return k(table, idx2d)


def _sc_scatter_add(msg_flat, recv2d):
    per_tile = _E // 16
    nb = per_tile // _GB
    nrow_t = _N // 16

    def _zero_acc(zsrc, acc, nbase):
        for q in range(7):
            pltpu.sync_copy(zsrc, acc.at[pl.ds(nbase + q * _GB, _GB)])
        pltpu.sync_copy(zsrc.at[pl.ds(0, 65)], acc.at[pl.ds(nbase + 7 * _GB, 65)])

    @functools.partial(
        pl.kernel,
        mesh=_sc_mesh(),
        out_type=jax.ShapeDtypeStruct((4 * _N, 128), jnp.float32),
        scratch_types=[
            pltpu.VMEM((nb, _GB), jnp.int32),
            pltpu.VMEM((_GB, 128), jnp.float32),
            pltpu.VMEM((_GB, 128), jnp.float32),
            pltpu.VMEM_SHARED((_N, 128), jnp.float32),
            pltpu.SemaphoreType.DMA,
            pltpu.SemaphoreType.DMA,
        ],
        compiler_params=pltpu.CompilerParams(use_tc_tiling_on_sc=False),
    )
    def k(msg_hbm, idx_hbm, zer_hbm, out_hbm, idx_v, buf0, buf1, acc, sem0, sem1):
        cid = lax.axis_index("c")
        sid = lax.axis_index("s")
        ebase = sid * per_tile
        nbase = sid * nrow_t
        pltpu.sync_copy(idx_hbm.at[pl.ds(sid * nb, nb)], idx_v)
        pltpu.sync_copy(zer_hbm, buf0)
        _zero_acc(buf0, acc, nbase)
        plsc.subcore_barrier()
        for chunk in range(2):
            cglob = cid * 2 + chunk
            row0 = cglob * _E + ebase

            def body(j, _):
                pltpu.async_copy(
                    msg_hbm.at[pl.ds(row0 + j * _GB, _GB)], buf1, sem1
                ).wait()
                pltpu.sync_copy(buf1, acc.at[idx_v.at[j]], add=True)
                return 0

            lax.fori_loop(0, nb, body, 0)
            plsc.subcore_barrier()
            pltpu.sync_copy(
                acc.at[pl.ds(nbase, nrow_t)],
                out_hbm.at[pl.ds(cglob * _N + nbase, nrow_t)],
            )
            if chunk == 0:
                pltpu.sync_copy(zer_hbm, buf0)
                _zero_acc(buf0, acc, nbase)
                plsc.subcore_barrier()

    zer = jnp.zeros((_GB, 128), jnp.float32)
    return k(msg_flat, recv2d, zer)


def _make_mw(w_upd_t):
    rows = np.zeros((4, 128), np.int32)
    for k in range(4):
        for c8 in range(8):
            for lm in range(_NLM):
                rows[k, c8 * 16 + lm] = _L_OF[lm] * _C + 8 * k + c8
    return w_upd_t[rows.reshape(-1)].reshape(4, 128, _C)


def kernel(positions, species, senders, receivers, species_embed, W_rad1, W_rad2, W_upd, W_read):
    pos_pad = jnp.pad(positions, ((0, 0), (0, 13)))
    send2d = senders.astype(jnp.int32).reshape(_E // _GB, _GB)
    recv2d = receivers.astype(jnp.int32).reshape(_E // _GB, _GB)

    ps = _sc_gather(pos_pad, send2d, _E, 16)
    pr = _sc_gather(pos_pad, recv2d, _E, 16)
    rb, sh = _geom(ps, pr)

    npad = 10240
    spec2d = jnp.pad(species.astype(jnp.int32), (0, npad - _N)).reshape(npad // _GB, _GB)
    h = _sc_gather(species_embed, spec2d, npad, _C)[:_N]

    q = _make_q()
    reads = []
    for t in range(_T):
        hs = _sc_gather(h, send2d, _E, _C)
        msg4 = _edge(rb, sh, hs, W_rad1[t], _edge_weight_prep(W_rad2[t]), q)
        agg4 = _sc_scatter_add(msg4.reshape(4 * _E, 128), recv2d).reshape(4, _N, 128)
        h, rd = _node(agg4, _make_mw(W_upd[t]), W_read[t])
        reads.append(rd)
    return jnp.concatenate(reads, axis=1)


# speedup vs baseline: 50.1156x; 1.1608x over previous
import functools

import numpy as np
import jax
import jax.numpy as jnp
from jax import lax
from jax.experimental import pallas as pl
from jax.experimental.pallas import tpu as pltpu
from jax.experimental.pallas import tpu_sc as plsc

_N = 10000
_E = 320000
_C = 32
_NBAS = 8
_RMAX = 5.0
_LMAX = 3
_T = 2
_OUT = 128
_HID = 64
_AVG = 32.0
_NLM = 16

_L_OF = np.array([0, 1, 1, 1, 2, 2, 2, 2, 2, 3, 3, 3, 3, 3, 3, 3], np.int32)

_EB = 512
_NB = 1000


def _silu(x):
    return x / (1.0 + jnp.exp(-x))


_EBG = 2560


def _geom_body(psT_ref, prT_ref, rbT_ref, shT_ref):
    relT = prT_ref[...] - psT_ref[...]
    x = relT[0:1, :]
    y = relT[1:2, :]
    z = relT[2:3, :]
    d2 = x * x + y * y + z * z
    dist = jnp.sqrt(d2)
    inv = 1.0 / jnp.maximum(dist, 1e-6)
    x = x * inv
    y = y * inv
    z = z * inv

    ks = (jnp.arange(_NBAS, dtype=jnp.int32).astype(jnp.float32) + 1.0)[:, None]
    d_safe = jnp.where(dist == 0.0, 1e-6, dist)
    rb = jnp.sqrt(2.0 / _RMAX) * jnp.sin(ks * ((jnp.pi / _RMAX) * d_safe)) / d_safe
    u = dist * (1.0 / _RMAX)
    u2 = u * u
    env = jnp.exp(-u2 / jnp.clip(1.0 - u2, 1e-6, None))
    env = jnp.where(u < 1.0, env, 0.0)
    rbT_ref[...] = rb * env

    one = jnp.ones_like(x)
    x2 = x * x
    y2 = y * y
    z2 = z * z
    rows = [
        0.28209479 * one,
        0.48860251 * y,
        0.48860251 * z,
        0.48860251 * x,
        1.09254843 * x * y,
        1.09254843 * y * z,
        0.31539157 * (3.0 * z2 - 1.0),
        1.09254843 * x * z,
        0.54627422 * (x2 - y2),
        0.59004359 * y * (3.0 * x2 - y2),
        2.89061144 * x * y * z,
        0.45704580 * y * (5.0 * z2 - 1.0),
        0.37317633 * z * (5.0 * z2 - 3.0),
        0.45704580 * x * (5.0 * z2 - 1.0),
        1.44530572 * z * (x2 - y2),
        0.59004359 * x * (x2 - 3.0 * y2),
    ]
    shT_ref[...] = jnp.concatenate(rows, axis=0)


def _geom(psT, prT):
    grid = _E // _EBG
    return pl.pallas_call(
        _geom_body,
        grid=(grid,),
        in_specs=[
            pl.BlockSpec((16, _EBG), lambda i: (0, i)),
            pl.BlockSpec((16, _EBG), lambda i: (0, i)),
        ],
        out_specs=[
            pl.BlockSpec((_NBAS, _EBG), lambda i: (0, i)),
            pl.BlockSpec((_NLM, _EBG), lambda i: (0, i)),
        ],
        out_shape=[
            jax.ShapeDtypeStruct((_NBAS, _E), jnp.float32),
            jax.ShapeDtypeStruct((_NLM, _E), jnp.float32),
        ],
    )(psT, prT)


_EBE = 1280


def _edge_body(rbT_ref, shT_ref, hs_ref, w1_ref, w2p_ref, q_ref, r4_ref, out_ref):
    rbT = rbT_ref[...]
    shT = shT_ref[...]
    hs = hs_ref[...]
    hidT = _silu(
        lax.dot_general(w1_ref[...], rbT, (((0,), (0,)), ((), ())),
                        preferred_element_type=jnp.float32)
    )
    radlm = lax.dot_general(hidT, w2p_ref[...], (((0,), (0,)), ((), ())),
                            preferred_element_type=jnp.float32)
    hsx = jnp.dot(hs, q_ref[...], preferred_element_type=jnp.float32)
    shx4 = lax.dot_general(shT, r4_ref[...], (((0,), (0,)), ((), ())),
                           preferred_element_type=jnp.float32)
    out_ref[...] = radlm * hsx * shx4


def _edge(rbT, shT, hs, w1, w2p, q, r4):
    grid = _E // _EBE
    return pl.pallas_call(
        _edge_body,
        grid=(grid,),
        in_specs=[
            pl.BlockSpec((_NBAS, _EBE), lambda i: (0, i)),
            pl.BlockSpec((_NLM, _EBE), lambda i: (0, i)),
            pl.BlockSpec((_EBE, _C), lambda i: (i, 0)),
            pl.BlockSpec((_NBAS, _HID), lambda i: (0, 0)),
            pl.BlockSpec((_HID, 512), lambda i: (0, 0)),
            pl.BlockSpec((_C, 512), lambda i: (0, 0)),
            pl.BlockSpec((_NLM, 512), lambda i: (0, 0)),
        ],
        out_specs=pl.BlockSpec((_EBE, 512), lambda i: (i, 0)),
        out_shape=jax.ShapeDtypeStruct((_E, 512), jnp.float32),
    )(rbT, shT, hs, w1, w2p, q, r4)


def _edge_weight_prep(w_rad2_t):
    colidx = np.zeros(512, np.int32)
    for k in range(4):
        for c8 in range(8):
            for lm in range(_NLM):
                colidx[128 * k + 16 * c8 + lm] = (8 * k + c8) * 4 + _L_OF[lm]
    return w_rad2_t[:, colidx]


def _make_q():
    q = np.zeros((_C, 512), np.float32)
    for k in range(4):
        for c8 in range(8):
            q[8 * k + c8, 128 * k + 16 * c8 : 128 * k + 16 * c8 + _NLM] = 1.0
    return jnp.asarray(q)


def _make_r4():
    r4 = np.zeros((_NLM, 512), np.float32)
    for j in range(512):
        r4[j % 16, j] = 1.0
    return jnp.asarray(r4)


def _node_body(agg_ref, mw_ref, wr_ref, h_ref, rd_ref):
    h_pre = jnp.zeros((_NB, _C), jnp.float32)
    for k in range(4):
        a = agg_ref[k] * (1.0 / _AVG)
        sq = a * a
        h_pre = h_pre + jnp.dot(sq, mw_ref[k], preferred_element_type=jnp.float32)
    h_new = _silu(h_pre)
    h_ref[...] = h_new
    rd_ref[...] = jnp.dot(h_new, wr_ref[...], preferred_element_type=jnp.float32)


def _node(agg4, mw, w_read):
    grid = _N // _NB
    return pl.pallas_call(
        _node_body,
        grid=(grid,),
        in_specs=[
            pl.BlockSpec((4, _NB, 128), lambda i: (0, i, 0)),
            pl.BlockSpec((4, 128, _C), lambda i: (0, 0, 0)),
            pl.BlockSpec((_C, _OUT), lambda i: (0, 0)),
        ],
        out_specs=[
            pl.BlockSpec((_NB, _C), lambda i: (i, 0)),
            pl.BlockSpec((_NB, _OUT), lambda i: (i, 0)),
        ],
        out_shape=[
            jax.ShapeDtypeStruct((_N, _C), jnp.float32),
            jax.ShapeDtypeStruct((_N, _OUT), jnp.float32),
        ],
    )(agg4, mw, w_read)


_GB = 80
_NW = 32


def _sc_mesh():
    return plsc.VectorSubcoreMesh(
        core_axis_name="c", subcore_axis_name="s", num_cores=2, num_subcores=16
    )


def _sc_gather(table, idx2d, rows, width):
    per_w = rows // _NW
    nb = per_w // _GB

    @functools.partial(
        pl.kernel,
        mesh=_sc_mesh(),
        out_type=jax.ShapeDtypeStruct((rows, width), jnp.float32),
        scratch_types=[
            pltpu.VMEM((nb, _GB), jnp.int32),
            pltpu.VMEM((_GB, width), jnp.float32),
            pltpu.VMEM((_GB, width), jnp.float32),
            pltpu.SemaphoreType.DMA,
            pltpu.SemaphoreType.DMA,
        ],
        compiler_params=pltpu.CompilerParams(use_tc_tiling_on_sc=False),
    )
    def k(table_hbm, idx_hbm, out_hbm, idx_v, buf0, buf1, sem0, sem1):
        cid = lax.axis_index("c")
        sid = lax.axis_index("s")
        wid = sid * 2 + cid
        base = wid * per_w
        pltpu.sync_copy(idx_hbm.at[pl.ds(wid * nb, nb)], idx_v)

        def body(j, _):
            pltpu.async_copy(table_hbm.at[idx_v.at[j]], buf0, sem0).wait()
            pltpu.sync_copy(buf0, out_hbm.at[pl.ds(base + j * _GB, _GB)])
            return 0

        lax.fori_loop(0, nb, body, 0)

    return k(table, idx2d)


def _sc_scatter_add(msg, recv2d):
    per_tile = _E // 16
    nb = per_tile // _GB
    nrow_t = _N // 16

    def _zero_acc(zsrc, acc, nbase):
        for q in range(7):
            pltpu.sync_copy(zsrc, acc.at[pl.ds(nbase + q * _GB, _GB)])
        pltpu.sync_copy(zsrc.at[pl.ds(0, 65)], acc.at[pl.ds(nbase + 7 * _GB, 65)])

    @functools.partial(
        pl.kernel,
        mesh=_sc_mesh(),
        out_type=jax.ShapeDtypeStruct((4, _N, 128), jnp.float32),
        scratch_types=[
            pltpu.VMEM((nb, _GB), jnp.int32),
            pltpu.VMEM((_GB, 128), jnp.float32),
            pltpu.VMEM((_GB, 128), jnp.float32),
            pltpu.VMEM_SHARED((_N, 128), jnp.float32),
            pltpu.SemaphoreType.DMA,
            pltpu.SemaphoreType.DMA,
        ],
        compiler_params=pltpu.CompilerParams(use_tc_tiling_on_sc=False),
    )
    def k(msg_hbm, idx_hbm, zer_hbm, out_hbm, idx_v, buf0, buf1, acc, sem0, sem1):
        cid = lax.axis_index("c")
        sid = lax.axis_index("s")
        ebase = sid * per_tile
        nbase = sid * nrow_t
        pltpu.sync_copy(idx_hbm.at[pl.ds(sid * nb, nb)], idx_v)
        pltpu.sync_copy(zer_hbm, buf0)
        _zero_acc(buf0, acc, nbase)
        plsc.subcore_barrier()
        for chunk in range(2):
            cglob = cid * 2 + chunk
            col0 = cglob * 128

            def body(j, _):
                pltpu.async_copy(
                    msg_hbm.at[pl.ds(ebase + j * _GB, _GB), pl.ds(col0, 128)],
                    buf1,
                    sem1,
                ).wait()
                pltpu.sync_copy(buf1, acc.at[idx_v.at[j]], add=True)
                return 0

            lax.fori_loop(0, nb, body, 0)
            plsc.subcore_barrier()
            pltpu.sync_copy(
                acc.at[pl.ds(nbase, nrow_t)],
                out_hbm.at[cglob, pl.ds(nbase, nrow_t)],
            )
            if chunk == 0:
                pltpu.sync_copy(zer_hbm, buf0)
                _zero_acc(buf0, acc, nbase)
                plsc.subcore_barrier()

    zer = jnp.zeros((_GB, 128), jnp.float32)
    return k(msg, recv2d, zer)


def _make_mw(w_upd_t):
    rows = np.zeros((4, 128), np.int32)
    for k in range(4):
        for c8 in range(8):
            for lm in range(_NLM):
                rows[k, c8 * 16 + lm] = _L_OF[lm] * _C + 8 * k + c8
    return w_upd_t[rows.reshape(-1)].reshape(4, 128, _C)


def kernel(positions, species, senders, receivers, species_embed, W_rad1, W_rad2, W_upd, W_read):
    pos_pad = jnp.pad(positions, ((0, 0), (0, 13)))
    send2d = senders.astype(jnp.int32).reshape(_E // _GB, _GB)
    recv2d = receivers.astype(jnp.int32).reshape(_E // _GB, _GB)

    ps = _sc_gather(pos_pad, send2d, _E, 16)
    pr = _sc_gather(pos_pad, recv2d, _E, 16)
    rbT, shT = _geom(ps.T, pr.T)

    npad = 10240
    spec2d = jnp.pad(species.astype(jnp.int32), (0, npad - _N)).reshape(npad // _GB, _GB)
    h = _sc_gather(species_embed, spec2d, npad, _C)[:_N]

    q = _make_q()
    r4 = _make_r4()
    reads = []
    for t in range(_T):
        hs = _sc_gather(h, send2d, _E, _C)
        msg = _edge(rbT, shT, hs, W_rad1[t], _edge_weight_prep(W_rad2[t]), q, r4)
        agg4 = _sc_scatter_add(msg, recv2d)
        h, rd = _node(agg4, _make_mw(W_upd[t]), W_read[t])
        reads.append(rd)
    return jnp.concatenate(reads, axis=1)


# speedup vs baseline: 58.8575x; 1.1744x over previous
import functools

import numpy as np
import jax
import jax.numpy as jnp
from jax import lax
from jax.experimental import pallas as pl
from jax.experimental.pallas import tpu as pltpu
from jax.experimental.pallas import tpu_sc as plsc

_N = 10000
_E = 320000
_C = 32
_NBAS = 8
_RMAX = 5.0
_LMAX = 3
_T = 2
_OUT = 128
_HID = 64
_AVG = 32.0
_NLM = 16

_L_OF = np.array([0, 1, 1, 1, 2, 2, 2, 2, 2, 3, 3, 3, 3, 3, 3, 3], np.int32)

_EB = 512
_NB = 1000


def _silu(x):
    return x / (1.0 + jnp.exp(-x))


_EBG = 2560


def _geom_body(ps_ref, pr_ref, rbT_ref, shT_ref):
    rel = pr_ref[...] - ps_ref[...]
    rr = lax.broadcasted_iota(jnp.int32, (16, 16), 0)
    cc = lax.broadcasted_iota(jnp.int32, (16, 16), 1)
    eye = jnp.where(rr == cc, 1.0, 0.0).astype(jnp.float32)
    relT = lax.dot_general(eye, rel, (((1,), (1,)), ((), ())),
                           preferred_element_type=jnp.float32)
    x = relT[0:1, :]
    y = relT[1:2, :]
    z = relT[2:3, :]
    d2 = x * x + y * y + z * z
    dist = jnp.sqrt(d2)
    inv = 1.0 / jnp.maximum(dist, 1e-6)
    x = x * inv
    y = y * inv
    z = z * inv

    ks = (jnp.arange(_NBAS, dtype=jnp.int32).astype(jnp.float32) + 1.0)[:, None]
    d_safe = jnp.where(dist == 0.0, 1e-6, dist)
    rb = jnp.sqrt(2.0 / _RMAX) * jnp.sin(ks * ((jnp.pi / _RMAX) * d_safe)) / d_safe
    u = dist * (1.0 / _RMAX)
    u2 = u * u
    env = jnp.exp(-u2 / jnp.clip(1.0 - u2, 1e-6, None))
    env = jnp.where(u < 1.0, env, 0.0)
    rbT_ref[...] = rb * env

    one = jnp.ones_like(x)
    x2 = x * x
    y2 = y * y
    z2 = z * z
    rows = [
        0.28209479 * one,
        0.48860251 * y,
        0.48860251 * z,
        0.48860251 * x,
        1.09254843 * x * y,
        1.09254843 * y * z,
        0.31539157 * (3.0 * z2 - 1.0),
        1.09254843 * x * z,
        0.54627422 * (x2 - y2),
        0.59004359 * y * (3.0 * x2 - y2),
        2.89061144 * x * y * z,
        0.45704580 * y * (5.0 * z2 - 1.0),
        0.37317633 * z * (5.0 * z2 - 3.0),
        0.45704580 * x * (5.0 * z2 - 1.0),
        1.44530572 * z * (x2 - y2),
        0.59004359 * x * (x2 - 3.0 * y2),
    ]
    shT_ref[...] = jnp.concatenate(rows, axis=0)


def _geom(psr):
    grid = _E // _EBG
    return pl.pallas_call(
        _geom_body,
        grid=(grid,),
        in_specs=[
            pl.BlockSpec((_EBG, 16), lambda i: (i, 0)),
            pl.BlockSpec((_EBG, 16), lambda i: (i + _E // _EBG, 0)),
        ],
        out_specs=[
            pl.BlockSpec((_NBAS, _EBG), lambda i: (0, i)),
            pl.BlockSpec((_NLM, _EBG), lambda i: (0, i)),
        ],
        out_shape=[
            jax.ShapeDtypeStruct((_NBAS, _E), jnp.float32),
            jax.ShapeDtypeStruct((_NLM, _E), jnp.float32),
        ],
    )(psr, psr)


_EBE = 1280


def _edge_body(rbT_ref, shT_ref, hs_ref, w1_ref, w2p_ref, q_ref, r4_ref, out_ref):
    rbT = rbT_ref[...]
    shT = shT_ref[...]
    hs = hs_ref[...]
    hidT = _silu(
        lax.dot_general(w1_ref[...], rbT, (((0,), (0,)), ((), ())),
                        preferred_element_type=jnp.float32)
    )
    radlm = lax.dot_general(hidT, w2p_ref[...], (((0,), (0,)), ((), ())),
                            preferred_element_type=jnp.float32)
    hsx = jnp.dot(hs, q_ref[...], preferred_element_type=jnp.float32)
    shx4 = lax.dot_general(shT, r4_ref[...], (((0,), (0,)), ((), ())),
                           preferred_element_type=jnp.float32)
    out_ref[...] = radlm * hsx * shx4


def _edge(rbT, shT, hs, w1, w2p, q, r4):
    grid = _E // _EBE
    return pl.pallas_call(
        _edge_body,
        grid=(grid,),
        in_specs=[
            pl.BlockSpec((_NBAS, _EBE), lambda i: (0, i)),
            pl.BlockSpec((_NLM, _EBE), lambda i: (0, i)),
            pl.BlockSpec((_EBE, _C), lambda i: (i, 0)),
            pl.BlockSpec((_NBAS, _HID), lambda i: (0, 0)),
            pl.BlockSpec((_HID, 512), lambda i: (0, 0)),
            pl.BlockSpec((_C, 512), lambda i: (0, 0)),
            pl.BlockSpec((_NLM, 512), lambda i: (0, 0)),
        ],
        out_specs=pl.BlockSpec((_EBE, 512), lambda i: (i, 0)),
        out_shape=jax.ShapeDtypeStruct((_E, 512), jnp.float32),
    )(rbT, shT, hs, w1, w2p, q, r4)


def _edge_weight_prep(w_rad2_t):
    colidx = np.zeros(512, np.int32)
    for k in range(4):
        for c8 in range(8):
            for lm in range(_NLM):
                colidx[128 * k + 16 * c8 + lm] = (8 * k + c8) * 4 + _L_OF[lm]
    return w_rad2_t[:, colidx]


def _make_q():
    q = np.zeros((_C, 512), np.float32)
    for k in range(4):
        for c8 in range(8):
            q[8 * k + c8, 128 * k + 16 * c8 : 128 * k + 16 * c8 + _NLM] = 1.0
    return jnp.asarray(q)


def _make_r4():
    r4 = np.zeros((_NLM, 512), np.float32)
    for j in range(512):
        r4[j % 16, j] = 1.0
    return jnp.asarray(r4)


def _node_body(agg_ref, mw_ref, wr_ref, h_ref, rd_ref):
    h_pre = jnp.zeros((_NB, _C), jnp.float32)
    for k in range(4):
        a = agg_ref[k] * (1.0 / _AVG)
        sq = a * a
        h_pre = h_pre + jnp.dot(sq, mw_ref[k], preferred_element_type=jnp.float32)
    h_new = _silu(h_pre)
    h_ref[...] = h_new
    rd_ref[...] = jnp.dot(h_new, wr_ref[...], preferred_element_type=jnp.float32)


def _node(agg4, mw, w_read):
    grid = _N // _NB
    return pl.pallas_call(
        _node_body,
        grid=(grid,),
        in_specs=[
            pl.BlockSpec((4, _NB, 128), lambda i: (0, i, 0)),
            pl.BlockSpec((4, 128, _C), lambda i: (0, 0, 0)),
            pl.BlockSpec((_C, _OUT), lambda i: (0, 0)),
        ],
        out_specs=[
            pl.BlockSpec((_NB, _C), lambda i: (i, 0)),
            pl.BlockSpec((_NB, _OUT), lambda i: (i, 0)),
        ],
        out_shape=[
            jax.ShapeDtypeStruct((_N, _C), jnp.float32),
            jax.ShapeDtypeStruct((_N, _OUT), jnp.float32),
        ],
    )(agg4, mw, w_read)


_GB = 80
_NW = 32


def _sc_mesh():
    return plsc.VectorSubcoreMesh(
        core_axis_name="c", subcore_axis_name="s", num_cores=2, num_subcores=16
    )


def _sc_gather(table, idx2d, rows, width):
    per_w = rows // _NW
    nb = per_w // _GB

    @functools.partial(
        pl.kernel,
        mesh=_sc_mesh(),
        out_type=jax.ShapeDtypeStruct((rows, width), jnp.float32),
        scratch_types=[
            pltpu.VMEM((nb, _GB), jnp.int32),
            pltpu.VMEM((_GB, width), jnp.float32),
            pltpu.VMEM((_GB, width), jnp.float32),
            pltpu.SemaphoreType.DMA,
            pltpu.SemaphoreType.DMA,
        ],
        compiler_params=pltpu.CompilerParams(use_tc_tiling_on_sc=False),
    )
    def k(table_hbm, idx_hbm, out_hbm, idx_v, buf0, buf1, sem0, sem1):
        cid = lax.axis_index("c")
        sid = lax.axis_index("s")
        wid = sid * 2 + cid
        base = wid * per_w
        pltpu.sync_copy(idx_hbm.at[pl.ds(wid * nb, nb)], idx_v)

        def body(j, _):
            pltpu.async_copy(table_hbm.at[idx_v.at[j]], buf0, sem0).wait()
            pltpu.sync_copy(buf0, out_hbm.at[pl.ds(base + j * _GB, _GB)])
            return 0

        lax.fori_loop(0, nb, body, 0)

    return k(table, idx2d)


def _sc_scatter_add(msg, recv2d):
    per_tile = _E // 16
    nb = per_tile // _GB
    nrow_t = _N // 16

    def _zero_acc(zsrc, acc, nbase):
        for q in range(7):
            pltpu.sync_copy(zsrc, acc.at[pl.ds(nbase + q * _GB, _GB)])
        pltpu.sync_copy(zsrc.at[pl.ds(0, 65)], acc.at[pl.ds(nbase + 7 * _GB, 65)])

    @functools.partial(
        pl.kernel,
        mesh=_sc_mesh(),
        out_type=jax.ShapeDtypeStruct((4, _N, 128), jnp.float32),
        scratch_types=[
            pltpu.VMEM((nb, _GB), jnp.int32),
            pltpu.VMEM((_GB, 128), jnp.float32),
            pltpu.VMEM((_GB, 128), jnp.float32),
            pltpu.VMEM_SHARED((_N, 128), jnp.float32),
            pltpu.SemaphoreType.DMA,
            pltpu.SemaphoreType.DMA,
        ],
        compiler_params=pltpu.CompilerParams(use_tc_tiling_on_sc=False),
    )
    def k(msg_hbm, idx_hbm, zer_hbm, out_hbm, idx_v, buf0, buf1, acc, sem0, sem1):
        cid = lax.axis_index("c")
        sid = lax.axis_index("s")
        ebase = sid * per_tile
        nbase = sid * nrow_t
        pltpu.sync_copy(idx_hbm.at[pl.ds(sid * nb, nb)], idx_v)
        pltpu.sync_copy(zer_hbm, buf0)
        _zero_acc(buf0, acc, nbase)
        plsc.subcore_barrier()
        bufs = (buf0, buf1)
        sems = (sem0, sem1)
        for chunk in range(2):
            cglob = cid * 2 + chunk
            col0 = cglob * 128

            def _src(j):
                return msg_hbm.at[pl.ds(ebase + j * _GB, _GB), pl.ds(col0, 128)]

            for b in range(2):
                pltpu.async_copy(_src(b), bufs[b], sems[b])

            def body(j2, _):
                j = j2 * 2
                for b in range(2):
                    jj = j + b
                    pltpu.make_async_copy(_src(jj), bufs[b], sems[b]).wait()
                    pltpu.sync_copy(bufs[b], acc.at[idx_v.at[jj]], add=True)

                    @pl.when(jj + 2 < nb)
                    def _():
                        pltpu.async_copy(_src(jj + 2), bufs[b], sems[b])

                return 0

            lax.fori_loop(0, nb // 2, body, 0)
            plsc.subcore_barrier()
            pltpu.sync_copy(
                acc.at[pl.ds(nbase, nrow_t)],
                out_hbm.at[cglob, pl.ds(nbase, nrow_t)],
            )
            if chunk == 0:
                pltpu.sync_copy(zer_hbm, buf0)
                _zero_acc(buf0, acc, nbase)
                plsc.subcore_barrier()

    zer = jnp.zeros((_GB, 128), jnp.float32)
    return k(msg, recv2d, zer)


def _make_mw(w_upd_t):
    rows = np.zeros((4, 128), np.int32)
    for k in range(4):
        for c8 in range(8):
            for lm in range(_NLM):
                rows[k, c8 * 16 + lm] = _L_OF[lm] * _C + 8 * k + c8
    return w_upd_t[rows.reshape(-1)].reshape(4, 128, _C)


def kernel(positions, species, senders, receivers, species_embed, W_rad1, W_rad2, W_upd, W_read):
    pos_pad = jnp.pad(positions, ((0, 0), (0, 13)))
    send2d = senders.astype(jnp.int32).reshape(_E // _GB, _GB)
    recv2d = receivers.astype(jnp.int32).reshape(_E // _GB, _GB)

    sr2d = jnp.concatenate([senders, receivers]).astype(jnp.int32).reshape(2 * _E // _GB, _GB)
    psr = _sc_gather(pos_pad, sr2d, 2 * _E, 16)
    rbT, shT = _geom(psr)

    npad = 10240
    spec2d = jnp.pad(species.astype(jnp.int32), (0, npad - _N)).reshape(npad // _GB, _GB)
    h = _sc_gather(species_embed, spec2d, npad, _C)[:_N]

    q = _make_q()
    r4 = _make_r4()
    reads = []
    for t in range(_T):
        hs = _sc_gather(h, send2d, _E, _C)
        msg = _edge(rbT, shT, hs, W_rad1[t], _edge_weight_prep(W_rad2[t]), q, r4)
        agg4 = _sc_scatter_add(msg, recv2d)
        h, rd = _node(agg4, _make_mw(W_upd[t]), W_read[t])
        reads.append(rd)
    return jnp.concatenate(reads, axis=1)


# speedup vs baseline: 59.4744x; 1.0105x over previous
import functools

import numpy as np
import jax
import jax.numpy as jnp
from jax import lax
from jax.experimental import pallas as pl
from jax.experimental.pallas import tpu as pltpu
from jax.experimental.pallas import tpu_sc as plsc

_N = 10000
_E = 320000
_C = 32
_NBAS = 8
_RMAX = 5.0
_LMAX = 3
_T = 2
_OUT = 128
_HID = 64
_AVG = 32.0
_NLM = 16

_L_OF = np.array([0, 1, 1, 1, 2, 2, 2, 2, 2, 3, 3, 3, 3, 3, 3, 3], np.int32)

_EB = 512
_NB = 1000


def _silu(x):
    return x / (1.0 + jnp.exp(-x))


_EBG = 2560


def _geom_body(ps_ref, pr_ref, rbT_ref, shT_ref):
    rel = pr_ref[...] - ps_ref[...]
    rr = lax.broadcasted_iota(jnp.int32, (16, 16), 0)
    cc = lax.broadcasted_iota(jnp.int32, (16, 16), 1)
    eye = jnp.where(rr == cc, 1.0, 0.0).astype(jnp.float32)
    relT = lax.dot_general(eye, rel, (((1,), (1,)), ((), ())),
                           preferred_element_type=jnp.float32)
    x = relT[0:1, :]
    y = relT[1:2, :]
    z = relT[2:3, :]
    d2 = x * x + y * y + z * z
    dist = jnp.sqrt(d2)
    inv = 1.0 / jnp.maximum(dist, 1e-6)
    x = x * inv
    y = y * inv
    z = z * inv

    ks = (jnp.arange(_NBAS, dtype=jnp.int32).astype(jnp.float32) + 1.0)[:, None]
    d_safe = jnp.where(dist == 0.0, 1e-6, dist)
    rb = jnp.sqrt(2.0 / _RMAX) * jnp.sin(ks * ((jnp.pi / _RMAX) * d_safe)) / d_safe
    u = dist * (1.0 / _RMAX)
    u2 = u * u
    env = jnp.exp(-u2 / jnp.clip(1.0 - u2, 1e-6, None))
    env = jnp.where(u < 1.0, env, 0.0)
    rbT_ref[...] = rb * env

    one = jnp.ones_like(x)
    x2 = x * x
    y2 = y * y
    z2 = z * z
    rows = [
        0.28209479 * one,
        0.48860251 * y,
        0.48860251 * z,
        0.48860251 * x,
        1.09254843 * x * y,
        1.09254843 * y * z,
        0.31539157 * (3.0 * z2 - 1.0),
        1.09254843 * x * z,
        0.54627422 * (x2 - y2),
        0.59004359 * y * (3.0 * x2 - y2),
        2.89061144 * x * y * z,
        0.45704580 * y * (5.0 * z2 - 1.0),
        0.37317633 * z * (5.0 * z2 - 3.0),
        0.45704580 * x * (5.0 * z2 - 1.0),
        1.44530572 * z * (x2 - y2),
        0.59004359 * x * (x2 - 3.0 * y2),
    ]
    shT_ref[...] = jnp.concatenate(rows, axis=0)


def _geom(psr):
    grid = _E // _EBG
    return pl.pallas_call(
        _geom_body,
        grid=(grid,),
        in_specs=[
            pl.BlockSpec((_EBG, 16), lambda i: (i, 0)),
            pl.BlockSpec((_EBG, 16), lambda i: (i + _E // _EBG, 0)),
        ],
        out_specs=[
            pl.BlockSpec((_NBAS, _EBG), lambda i: (0, i)),
            pl.BlockSpec((_NLM, _EBG), lambda i: (0, i)),
        ],
        out_shape=[
            jax.ShapeDtypeStruct((_NBAS, _E), jnp.float32),
            jax.ShapeDtypeStruct((_NLM, _E), jnp.float32),
        ],
    )(psr, psr)


_EBE = 1280


def _edge_body(rbT_ref, shT_ref, hs_ref, w1_ref, w2p_ref, q_ref, r4_ref, out_ref):
    rbT = rbT_ref[...]
    shT = shT_ref[...]
    hs = hs_ref[...]
    hidT = _silu(
        lax.dot_general(w1_ref[...], rbT, (((0,), (0,)), ((), ())),
                        preferred_element_type=jnp.float32)
    )
    radlm = lax.dot_general(hidT, w2p_ref[...], (((0,), (0,)), ((), ())),
                            preferred_element_type=jnp.float32)
    hsx = jnp.dot(hs, q_ref[...], preferred_element_type=jnp.float32)
    shx4 = lax.dot_general(shT, r4_ref[...], (((0,), (0,)), ((), ())),
                           preferred_element_type=jnp.float32)
    out_ref[...] = radlm * hsx * shx4


def _edge(rbT, shT, hs, w1, w2p, q, r4):
    grid = _E // _EBE
    return pl.pallas_call(
        _edge_body,
        grid=(grid,),
        in_specs=[
            pl.BlockSpec((_NBAS, _EBE), lambda i: (0, i)),
            pl.BlockSpec((_NLM, _EBE), lambda i: (0, i)),
            pl.BlockSpec((_EBE, _C), lambda i: (i, 0)),
            pl.BlockSpec((_NBAS, _HID), lambda i: (0, 0)),
            pl.BlockSpec((_HID, 512), lambda i: (0, 0)),
            pl.BlockSpec((_C, 512), lambda i: (0, 0)),
            pl.BlockSpec((_NLM, 512), lambda i: (0, 0)),
        ],
        out_specs=pl.BlockSpec((_EBE, 512), lambda i: (i, 0)),
        out_shape=jax.ShapeDtypeStruct((_E, 512), jnp.float32),
    )(rbT, shT, hs, w1, w2p, q, r4)


def _edge_weight_prep(w_rad2_t):
    colidx = np.zeros(512, np.int32)
    for k in range(4):
        for c8 in range(8):
            for lm in range(_NLM):
                colidx[128 * k + 16 * c8 + lm] = (8 * k + c8) * 4 + _L_OF[lm]
    return w_rad2_t[:, colidx]


def _make_q():
    q = np.zeros((_C, 512), np.float32)
    for k in range(4):
        for c8 in range(8):
            q[8 * k + c8, 128 * k + 16 * c8 : 128 * k + 16 * c8 + _NLM] = 1.0
    return jnp.asarray(q)


def _make_r4():
    r4 = np.zeros((_NLM, 512), np.float32)
    for j in range(512):
        r4[j % 16, j] = 1.0
    return jnp.asarray(r4)


def _node_body(agg_ref, mw_ref, wr_ref, h_ref, rd_ref):
    h_pre = jnp.zeros((_NB, _C), jnp.float32)
    for k in range(4):
        a = agg_ref[k] * (1.0 / _AVG)
        sq = a * a
        h_pre = h_pre + jnp.dot(sq, mw_ref[k], preferred_element_type=jnp.float32)
    h_new = _silu(h_pre)
    h_ref[...] = h_new
    rd_ref[...] = jnp.dot(h_new, wr_ref[...], preferred_element_type=jnp.float32)


def _node(agg4, mw, w_read):
    grid = _N // _NB
    return pl.pallas_call(
        _node_body,
        grid=(grid,),
        in_specs=[
            pl.BlockSpec((4, _NB, 128), lambda i: (0, i, 0)),
            pl.BlockSpec((4, 128, _C), lambda i: (0, 0, 0)),
            pl.BlockSpec((_C, _OUT), lambda i: (0, 0)),
        ],
        out_specs=[
            pl.BlockSpec((_NB, _C), lambda i: (i, 0)),
            pl.BlockSpec((_NB, _OUT), lambda i: (i, 0)),
        ],
        out_shape=[
            jax.ShapeDtypeStruct((_N, _C), jnp.float32),
            jax.ShapeDtypeStruct((_N, _OUT), jnp.float32),
        ],
    )(agg4, mw, w_read)


_GB = 80
_NW = 32


def _sc_mesh():
    return plsc.VectorSubcoreMesh(
        core_axis_name="c", subcore_axis_name="s", num_cores=2, num_subcores=16
    )


def _sc_gather(table, idx2d, rows, width):
    per_w = rows // _NW
    nb = per_w // _GB

    @functools.partial(
        pl.kernel,
        mesh=_sc_mesh(),
        out_type=jax.ShapeDtypeStruct((rows, width), jnp.float32),
        scratch_types=[
            pltpu.VMEM((nb, _GB), jnp.int32),
            pltpu.VMEM((_GB, width), jnp.float32),
            pltpu.VMEM((_GB, width), jnp.float32),
            pltpu.SemaphoreType.DMA,
            pltpu.SemaphoreType.DMA,
            pltpu.SemaphoreType.DMA,
        ],
        compiler_params=pltpu.CompilerParams(use_tc_tiling_on_sc=False),
    )
    def k(table_hbm, idx_hbm, out_hbm, idx_v, buf0, buf1, sem0, sem1, gsem):
        cid = lax.axis_index("c")
        sid = lax.axis_index("s")
        wid = sid * 2 + cid
        base = wid * per_w
        pltpu.sync_copy(idx_hbm.at[pl.ds(wid * nb, nb)], idx_v)
        bufs = (buf0, buf1)
        sems = (sem0, sem1)

        def _dst(j):
            return out_hbm.at[pl.ds(base + j * _GB, _GB)]

        def body(j2, _):
            j = j2 * 2
            for b in range(2):
                jj = j + b

                @pl.when(jj >= 2)
                def _():
                    pltpu.make_async_copy(bufs[b], _dst(jj - 2), sems[b]).wait()

                pltpu.async_copy(table_hbm.at[idx_v.at[jj]], bufs[b], gsem).wait()
                pltpu.async_copy(bufs[b], _dst(jj), sems[b])
            return 0

        lax.fori_loop(0, nb // 2, body, 0)
        if nb % 2 == 1:
            jj = nb - 1
            pltpu.make_async_copy(bufs[0], _dst(jj - 2), sems[0]).wait()
            pltpu.async_copy(table_hbm.at[idx_v.at[jj]], bufs[0], gsem).wait()
            pltpu.async_copy(bufs[0], _dst(jj), sems[0])
            pltpu.make_async_copy(bufs[1], _dst(nb - 2), sems[1]).wait()
            pltpu.make_async_copy(bufs[0], _dst(nb - 1), sems[0]).wait()
        else:
            for b in range(2):
                jj = nb - 2 + b
                pltpu.make_async_copy(bufs[b], _dst(jj), sems[b]).wait()

    return k(table, idx2d)


def _sc_scatter_add(msg, recv2d):
    per_tile = _E // 16
    nb = per_tile // _GB
    nrow_t = _N // 16

    def _zero_acc(zsrc, acc, nbase):
        for q in range(7):
            pltpu.sync_copy(zsrc, acc.at[pl.ds(nbase + q * _GB, _GB)])
        pltpu.sync_copy(zsrc.at[pl.ds(0, 65)], acc.at[pl.ds(nbase + 7 * _GB, 65)])

    @functools.partial(
        pl.kernel,
        mesh=_sc_mesh(),
        out_type=jax.ShapeDtypeStruct((4, _N, 128), jnp.float32),
        scratch_types=[
            pltpu.VMEM((nb, _GB), jnp.int32),
            pltpu.VMEM((_GB, 128), jnp.float32),
            pltpu.VMEM((_GB, 128), jnp.float32),
            pltpu.VMEM_SHARED((_N, 128), jnp.float32),
            pltpu.SemaphoreType.DMA,
            pltpu.SemaphoreType.DMA,
        ],
        compiler_params=pltpu.CompilerParams(use_tc_tiling_on_sc=False),
    )
    def k(msg_hbm, idx_hbm, zer_hbm, out_hbm, idx_v, buf0, buf1, acc, sem0, sem1):
        cid = lax.axis_index("c")
        sid = lax.axis_index("s")
        ebase = sid * per_tile
        nbase = sid * nrow_t
        pltpu.sync_copy(idx_hbm.at[pl.ds(sid * nb, nb)], idx_v)
        pltpu.sync_copy(zer_hbm, buf0)
        _zero_acc(buf0, acc, nbase)
        plsc.subcore_barrier()
        bufs = (buf0, buf1)
        sems = (sem0, sem1)
        for chunk in range(2):
            cglob = cid * 2 + chunk
            col0 = cglob * 128

            def _src(j):
                return msg_hbm.at[pl.ds(ebase + j * _GB, _GB), pl.ds(col0, 128)]

            for b in range(2):
                pltpu.async_copy(_src(b), bufs[b], sems[b])

            def body(j2, _):
                j = j2 * 2
                for b in range(2):
                    jj = j + b
                    pltpu.make_async_copy(_src(jj), bufs[b], sems[b]).wait()
                    pltpu.sync_copy(bufs[b], acc.at[idx_v.at[jj]], add=True)

                    @pl.when(jj + 2 < nb)
                    def _():
                        pltpu.async_copy(_src(jj + 2), bufs[b], sems[b])

                return 0

            lax.fori_loop(0, nb // 2, body, 0)
            plsc.subcore_barrier()
            pltpu.sync_copy(
                acc.at[pl.ds(nbase, nrow_t)],
                out_hbm.at[cglob, pl.ds(nbase, nrow_t)],
            )
            if chunk == 0:
                pltpu.sync_copy(zer_hbm, buf0)
                _zero_acc(buf0, acc, nbase)
                plsc.subcore_barrier()

    zer = jnp.zeros((_GB, 128), jnp.float32)
    return k(msg, recv2d, zer)


def _make_mw(w_upd_t):
    rows = np.zeros((4, 128), np.int32)
    for k in range(4):
        for c8 in range(8):
            for lm in range(_NLM):
                rows[k, c8 * 16 + lm] = _L_OF[lm] * _C + 8 * k + c8
    return w_upd_t[rows.reshape(-1)].reshape(4, 128, _C)


def kernel(positions, species, senders, receivers, species_embed, W_rad1, W_rad2, W_upd, W_read):
    pos_pad = jnp.pad(positions, ((0, 0), (0, 13)))
    send2d = senders.astype(jnp.int32).reshape(_E // _GB, _GB)
    recv2d = receivers.astype(jnp.int32).reshape(_E // _GB, _GB)

    sr2d = jnp.concatenate([senders, receivers]).astype(jnp.int32).reshape(2 * _E // _GB, _GB)
    psr = _sc_gather(pos_pad, sr2d, 2 * _E, 16)
    rbT, shT = _geom(psr)

    npad = 10240
    spec2d = jnp.pad(species.astype(jnp.int32), (0, npad - _N)).reshape(npad // _GB, _GB)
    h = _sc_gather(species_embed, spec2d, npad, _C)[:_N]

    q = _make_q()
    r4 = _make_r4()
    reads = []
    for t in range(_T):
        hs = _sc_gather(h, send2d, _E, _C)
        msg = _edge(rbT, shT, hs, W_rad1[t], _edge_weight_prep(W_rad2[t]), q, r4)
        agg4 = _sc_scatter_add(msg, recv2d)
        h, rd = _node(agg4, _make_mw(W_upd[t]), W_read[t])
        reads.append(rd)
    return jnp.concatenate(reads, axis=1)


# speedup vs baseline: 59.5075x; 1.0006x over previous
import functools

import numpy as np
import jax
import jax.numpy as jnp
from jax import lax
from jax.experimental import pallas as pl
from jax.experimental.pallas import tpu as pltpu
from jax.experimental.pallas import tpu_sc as plsc

_N = 10000
_E = 320000
_C = 32
_NBAS = 8
_RMAX = 5.0
_LMAX = 3
_T = 2
_OUT = 128
_HID = 64
_AVG = 32.0
_NLM = 16

_L_OF = np.array([0, 1, 1, 1, 2, 2, 2, 2, 2, 3, 3, 3, 3, 3, 3, 3], np.int32)

_NB = 1000


def _silu(x):
    return x / (1.0 + jnp.exp(-x))


_EBG = 2560


def _geom_body(ps_ref, pr_ref, rbT_ref, shT_ref):
    rel = pr_ref[...] - ps_ref[...]
    rr = lax.broadcasted_iota(jnp.int32, (16, 16), 0)
    cc = lax.broadcasted_iota(jnp.int32, (16, 16), 1)
    eye = jnp.where(rr == cc, 1.0, 0.0).astype(jnp.float32)
    relT = lax.dot_general(eye, rel, (((1,), (1,)), ((), ())),
                           preferred_element_type=jnp.float32)
    x = relT[0:1, :]
    y = relT[1:2, :]
    z = relT[2:3, :]
    d2 = x * x + y * y + z * z
    dist = jnp.sqrt(d2)
    inv = 1.0 / jnp.maximum(dist, 1e-6)
    x = x * inv
    y = y * inv
    z = z * inv

    ks = (jnp.arange(_NBAS, dtype=jnp.int32).astype(jnp.float32) + 1.0)[:, None]
    d_safe = jnp.where(dist == 0.0, 1e-6, dist)
    rb = jnp.sqrt(2.0 / _RMAX) * jnp.sin(ks * ((jnp.pi / _RMAX) * d_safe)) / d_safe
    u = dist * (1.0 / _RMAX)
    u2 = u * u
    env = jnp.exp(-u2 / jnp.clip(1.0 - u2, 1e-6, None))
    env = jnp.where(u < 1.0, env, 0.0)
    rbT_ref[...] = rb * env

    one = jnp.ones_like(x)
    x2 = x * x
    y2 = y * y
    z2 = z * z
    rows = [
        0.28209479 * one,
        0.48860251 * y,
        0.48860251 * z,
        0.48860251 * x,
        1.09254843 * x * y,
        1.09254843 * y * z,
        0.31539157 * (3.0 * z2 - 1.0),
        1.09254843 * x * z,
        0.54627422 * (x2 - y2),
        0.59004359 * y * (3.0 * x2 - y2),
        2.89061144 * x * y * z,
        0.45704580 * y * (5.0 * z2 - 1.0),
        0.37317633 * z * (5.0 * z2 - 3.0),
        0.45704580 * x * (5.0 * z2 - 1.0),
        1.44530572 * z * (x2 - y2),
        0.59004359 * x * (x2 - 3.0 * y2),
    ]
    shT_ref[...] = jnp.concatenate(rows, axis=0)


def _geom(psr):
    grid = _E // _EBG
    return pl.pallas_call(
        _geom_body,
        grid=(grid,),
        in_specs=[
            pl.BlockSpec((_EBG, 16), lambda i: (i, 0)),
            pl.BlockSpec((_EBG, 16), lambda i: (i + _E // _EBG, 0)),
        ],
        out_specs=[
            pl.BlockSpec((_NBAS, _EBG), lambda i: (0, i)),
            pl.BlockSpec((_NLM, _EBG), lambda i: (0, i)),
        ],
        out_shape=[
            jax.ShapeDtypeStruct((_NBAS, _E), jnp.float32),
            jax.ShapeDtypeStruct((_NLM, _E), jnp.float32),
        ],
    )(psr, psr)


_EBE = 1280


def _edge_body(rbT_ref, shT_ref, hs_ref, w1_ref, w2p_ref, q_ref, r4_ref, out_ref):
    rbT = rbT_ref[...]
    shT = shT_ref[...]
    hs = hs_ref[...]
    hidT = _silu(
        lax.dot_general(w1_ref[...], rbT, (((0,), (0,)), ((), ())),
                        preferred_element_type=jnp.float32)
    )
    radlm = lax.dot_general(hidT, w2p_ref[...], (((0,), (0,)), ((), ())),
                            preferred_element_type=jnp.float32)
    hsx = jnp.dot(hs, q_ref[...], preferred_element_type=jnp.float32)
    shx4 = lax.dot_general(shT, r4_ref[...], (((0,), (0,)), ((), ())),
                           preferred_element_type=jnp.float32)
    out_ref[...] = radlm * hsx * shx4


def _edge(rbT, shT, hs, w1, w2p, q, r4):
    grid = _E // _EBE
    return pl.pallas_call(
        _edge_body,
        grid=(grid,),
        in_specs=[
            pl.BlockSpec((_NBAS, _EBE), lambda i: (0, i)),
            pl.BlockSpec((_NLM, _EBE), lambda i: (0, i)),
            pl.BlockSpec((_EBE, _C), lambda i: (i, 0)),
            pl.BlockSpec((_NBAS, _HID), lambda i: (0, 0)),
            pl.BlockSpec((_HID, 512), lambda i: (0, 0)),
            pl.BlockSpec((_C, 512), lambda i: (0, 0)),
            pl.BlockSpec((_NLM, 512), lambda i: (0, 0)),
        ],
        out_specs=pl.BlockSpec((_EBE, 512), lambda i: (i, 0)),
        out_shape=jax.ShapeDtypeStruct((_E, 512), jnp.float32),
    )(rbT, shT, hs, w1, w2p, q, r4)


def _edge_weight_prep(w_rad2_t):
    colidx = np.zeros(512, np.int32)
    for k in range(4):
        for c8 in range(8):
            for lm in range(_NLM):
                colidx[128 * k + 16 * c8 + lm] = (8 * k + c8) * 4 + _L_OF[lm]
    return w_rad2_t[:, colidx]


def _make_q():
    q = np.zeros((_C, 512), np.float32)
    for k in range(4):
        for c8 in range(8):
            q[8 * k + c8, 128 * k + 16 * c8 : 128 * k + 16 * c8 + _NLM] = 1.0
    return jnp.asarray(q)


def _make_r4():
    r4 = np.zeros((_NLM, 512), np.float32)
    for j in range(512):
        r4[j % 16, j] = 1.0
    return jnp.asarray(r4)


def _node_body(agg_ref, mw_ref, wr_ref, h_ref, rd_ref):
    h_pre = jnp.zeros((_NB, _C), jnp.float32)
    for k in range(4):
        a = agg_ref[k] * (1.0 / _AVG)
        sq = a * a
        h_pre = h_pre + jnp.dot(sq, mw_ref[k], preferred_element_type=jnp.float32)
    h_new = _silu(h_pre)
    h_ref[...] = h_new
    rd_ref[...] = jnp.dot(h_new, wr_ref[...], preferred_element_type=jnp.float32)


def _node(agg4, mw, w_read):
    grid = _N // _NB
    return pl.pallas_call(
        _node_body,
        grid=(grid,),
        in_specs=[
            pl.BlockSpec((4, _NB, 128), lambda i: (0, i, 0)),
            pl.BlockSpec((4, 128, _C), lambda i: (0, 0, 0)),
            pl.BlockSpec((_C, _OUT), lambda i: (0, 0)),
        ],
        out_specs=[
            pl.BlockSpec((_NB, _C), lambda i: (i, 0)),
            pl.BlockSpec((_NB, _OUT), lambda i: (i, 0)),
        ],
        out_shape=[
            jax.ShapeDtypeStruct((_N, _C), jnp.float32),
            jax.ShapeDtypeStruct((_N, _OUT), jnp.float32),
        ],
    )(agg4, mw, w_read)


_GB = 80
_NW = 32


def _sc_mesh():
    return plsc.VectorSubcoreMesh(
        core_axis_name="c", subcore_axis_name="s", num_cores=2, num_subcores=16
    )


def _sc_gather(table, idx2d, rows, width):
    per_w = rows // _NW
    nb = per_w // _GB

    @functools.partial(
        pl.kernel,
        mesh=_sc_mesh(),
        out_type=jax.ShapeDtypeStruct((rows, width), jnp.float32),
        scratch_types=[
            pltpu.VMEM((nb, _GB), jnp.int32),
            pltpu.VMEM((_GB, width), jnp.float32),
            pltpu.VMEM((_GB, width), jnp.float32),
            pltpu.SemaphoreType.DMA,
            pltpu.SemaphoreType.DMA,
            pltpu.SemaphoreType.DMA,
        ],
        compiler_params=pltpu.CompilerParams(use_tc_tiling_on_sc=False),
    )
    def k(table_hbm, idx_hbm, out_hbm, idx_v, buf0, buf1, sem0, sem1, gsem):
        cid = lax.axis_index("c")
        sid = lax.axis_index("s")
        wid = sid * 2 + cid
        base = wid * per_w
        pltpu.sync_copy(idx_hbm.at[pl.ds(wid * nb, nb)], idx_v)
        bufs = (buf0, buf1)
        sems = (sem0, sem1)

        def _dst(j):
            return out_hbm.at[pl.ds(base + j * _GB, _GB)]

        def body(j2, _):
            j = j2 * 2
            for b in range(2):
                jj = j + b

                @pl.when(jj >= 2)
                def _():
                    pltpu.make_async_copy(bufs[b], _dst(jj - 2), sems[b]).wait()

                pltpu.async_copy(table_hbm.at[idx_v.at[jj]], bufs[b], gsem).wait()
                pltpu.async_copy(bufs[b], _dst(jj), sems[b])
            return 0

        lax.fori_loop(0, nb // 2, body, 0)
        if nb % 2 == 1:
            jj = nb - 1
            pltpu.make_async_copy(bufs[0], _dst(jj - 2), sems[0]).wait()
            pltpu.async_copy(table_hbm.at[idx_v.at[jj]], bufs[0], gsem).wait()
            pltpu.async_copy(bufs[0], _dst(jj), sems[0])
            pltpu.make_async_copy(bufs[1], _dst(nb - 2), sems[1]).wait()
            pltpu.make_async_copy(bufs[0], _dst(nb - 1), sems[0]).wait()
        else:
            for b in range(2):
                jj = nb - 2 + b
                pltpu.make_async_copy(bufs[b], _dst(jj), sems[b]).wait()

    return k(table, idx2d)


def _sc_scatter_add(msg, recv2d):
    per_tile = _E // 16
    nb = per_tile // _GB
    nrow_t = _N // 16

    def _zero_acc(zsrc, acc, nbase):
        for q in range(7):
            pltpu.sync_copy(zsrc, acc.at[pl.ds(nbase + q * _GB, _GB)])
        pltpu.sync_copy(zsrc.at[pl.ds(0, 65)], acc.at[pl.ds(nbase + 7 * _GB, 65)])

    @functools.partial(
        pl.kernel,
        mesh=_sc_mesh(),
        out_type=jax.ShapeDtypeStruct((4, _N, 128), jnp.float32),
        scratch_types=[
            pltpu.VMEM((nb, _GB), jnp.int32),
            pltpu.VMEM((_GB, 128), jnp.float32),
            pltpu.VMEM((_GB, 128), jnp.float32),
            pltpu.VMEM_SHARED((_N, 128), jnp.float32),
            pltpu.SemaphoreType.DMA,
            pltpu.SemaphoreType.DMA,
        ],
        compiler_params=pltpu.CompilerParams(use_tc_tiling_on_sc=False),
    )
    def k(msg_hbm, idx_hbm, zer_hbm, out_hbm, idx_v, buf0, buf1, acc, sem0, sem1):
        cid = lax.axis_index("c")
        sid = lax.axis_index("s")
        ebase = sid * per_tile
        nbase = sid * nrow_t
        pltpu.sync_copy(idx_hbm.at[pl.ds(sid * nb, nb)], idx_v)
        pltpu.sync_copy(zer_hbm, buf0)
        _zero_acc(buf0, acc, nbase)
        plsc.subcore_barrier()
        bufs = (buf0, buf1)
        sems = (sem0, sem1)
        for chunk in range(2):
            cglob = cid * 2 + chunk
            col0 = cglob * 128

            def _src(j):
                return msg_hbm.at[pl.ds(ebase + j * _GB, _GB), pl.ds(col0, 128)]

            for b in range(2):
                pltpu.async_copy(_src(b), bufs[b], sems[b])

            def body(j2, _):
                j = j2 * 2
                for b in range(2):
                    jj = j + b
                    pltpu.make_async_copy(_src(jj), bufs[b], sems[b]).wait()
                    pltpu.sync_copy(bufs[b], acc.at[idx_v.at[jj]], add=True)

                    @pl.when(jj + 2 < nb)
                    def _():
                        pltpu.async_copy(_src(jj + 2), bufs[b], sems[b])

                return 0

            lax.fori_loop(0, nb // 2, body, 0)
            plsc.subcore_barrier()
            pltpu.sync_copy(
                acc.at[pl.ds(nbase, nrow_t)],
                out_hbm.at[cglob, pl.ds(nbase, nrow_t)],
            )
            if chunk == 0:
                pltpu.sync_copy(zer_hbm, buf0)
                _zero_acc(buf0, acc, nbase)
                plsc.subcore_barrier()

    zer = jnp.zeros((_GB, 128), jnp.float32)
    return k(msg, recv2d, zer)


def _make_mw(w_upd_t):
    rows = np.zeros((4, 128), np.int32)
    for k in range(4):
        for c8 in range(8):
            for lm in range(_NLM):
                rows[k, c8 * 16 + lm] = _L_OF[lm] * _C + 8 * k + c8
    return w_upd_t[rows.reshape(-1)].reshape(4, 128, _C)


def kernel(positions, species, senders, receivers, species_embed, W_rad1, W_rad2, W_upd, W_read):
    pos_pad = jnp.pad(positions, ((0, 0), (0, 13)))
    send2d = senders.astype(jnp.int32).reshape(_E // _GB, _GB)
    recv2d = receivers.astype(jnp.int32).reshape(_E // _GB, _GB)

    sr2d = jnp.concatenate([senders, receivers]).astype(jnp.int32).reshape(2 * _E // _GB, _GB)
    psr = _sc_gather(pos_pad, sr2d, 2 * _E, 16)
    rbT, shT = _geom(psr)

    npad = 10240
    spec2d = jnp.pad(species.astype(jnp.int32), (0, npad - _N)).reshape(npad // _GB, _GB)
    h = _sc_gather(species_embed, spec2d, npad, _C)[:_N]

    q = _make_q()
    r4 = _make_r4()
    reads = []
    for t in range(_T):
        hs = _sc_gather(h, send2d, _E, _C)
        msg = _edge(rbT, shT, hs, W_rad1[t], _edge_weight_prep(W_rad2[t]), q, r4)
        agg4 = _sc_scatter_add(msg, recv2d)
        h, rd = _node(agg4, _make_mw(W_upd[t]), W_read[t])
        reads.append(rd)
    return jnp.concatenate(reads, axis=1)
